# Initial kernel scaffold; baseline (speedup 1.0000x reference)
import jax
import jax.numpy as jnp
from jax.experimental import pallas as pl


def kernel(preds0, preds1, preds2, gt_labels, anchors):
    raise NotImplementedError("write your pallas kernel here")


# speedup vs baseline: 2.4754x; 2.4754x over previous
import jax
import jax.numpy as jnp
from jax import lax
from jax.experimental import pallas as pl
from jax.experimental.pallas import tpu as pltpu

_B, _L, _C, _A = 32, 50, 80, 3
_IMG = 480.0
_R = 675
_SEGS = (16, 4, 1)
_NSEG = 21


def _level_body(x, g, anchors_ref, seg, W, loss_ref, boxes_ref, scores_ref):
    f32 = jnp.float32
    Wf = float(W)

    gx, gy, gw, gh, gc = (g[k:k + 1, :] for k in range(5))
    bw = gw * Wf
    bh = gh * Wf
    validg = bw > 0.0
    jg = jnp.clip(jnp.floor(gx * Wf), 0.0, Wf - 1.0)
    ig = jnp.clip(jnp.floor(gy * Wf), 0.0, Wf - 1.0)
    aw = [anchors_ref[k, 0] * Wf for k in range(_A)]
    ah = [anchors_ref[k, 1] * Wf for k in range(_A)]

    def anc_iou(k):
        inter = jnp.minimum(bw, aw[k]) * jnp.minimum(bh, ah[k])
        return inter / (bw * bh + aw[k] * ah[k] - inter + 1e-9)

    kb = jnp.zeros_like(gx)
    bestk = anc_iou(0)
    for k in (1, 2):
        iouk = anc_iou(k)
        upd = iouk > bestk
        kb = jnp.where(upd, float(k), kb)
        bestk = jnp.where(upd, iouk, bestk)
    anc_w = jnp.where(kb == 0.0, aw[0], jnp.where(kb == 1.0, aw[1], aw[2]))
    anc_h = jnp.where(kb == 0.0, ah[0], jnp.where(kb == 1.0, ah[1], ah[2]))
    bw_s = jnp.where(validg, bw, 1.0)
    bh_s = jnp.where(validg, bh, 1.0)
    adjx = gx * Wf - jg
    adjy = gy * Wf - ig
    adjw = jnp.log(bw_s / anc_w)
    adjh = jnp.log(bh_s / anc_h)
    jg_m = jnp.where(validg, jg, -1.0)

    rows = (lax.broadcasted_iota(jnp.int32, (_R, 1), 0).astype(f32)
            + seg.astype(f32) * float(_R))
    cell = jnp.floor((rows + 0.5) * (1.0 / 3.0))
    a = rows - 3.0 * cell
    i = jnp.floor((cell + 0.5) / Wf)
    j = cell - Wf * i

    tx = x[:, 0:1]
    ty = x[:, 1:2]
    tw = x[:, 2:3]
    th = x[:, 3:4]
    tc = x[:, 4:5]
    tcls = x[:, 5:5 + _C]

    sx = jax.nn.sigmoid(tx)
    sy = jax.nn.sigmoid(ty)
    aw_c = jnp.where(a == 0.0, aw[0], jnp.where(a == 1.0, aw[1], aw[2]))
    ah_c = jnp.where(a == 0.0, ah[0], jnp.where(a == 1.0, ah[1], ah[2]))
    px = (sx + j) / Wf
    py = (sy + i) / Wf
    pw = jnp.exp(tw) * aw_c / Wf
    ph = jnp.exp(th) * ah_c / Wf

    pminx = px - pw * 0.5
    pmaxx = px + pw * 0.5
    pminy = py - ph * 0.5
    pmaxy = py + ph * 0.5
    tminx = gx - gw * 0.5
    tmaxx = gx + gw * 0.5
    tminy = gy - gh * 0.5
    tmaxy = gy + gh * 0.5
    iw = jnp.clip(jnp.minimum(pmaxx, tmaxx) - jnp.maximum(pminx, tminx), 0.0)
    ih = jnp.clip(jnp.minimum(pmaxy, tmaxy) - jnp.maximum(pminy, tminy), 0.0)
    inter = iw * ih
    parea = pw * ph
    tarea = gw * gh
    iou = inter / (parea + tarea - inter + 1e-9)
    best = jnp.max(iou, axis=-1, keepdims=True)
    obj_det = (best > 0.6).astype(f32)

    m = (j == jg_m) & (i == ig) & (a == kb)
    liota = lax.broadcasted_iota(jnp.int32, (1, _L), 1).astype(f32)
    lsel = jnp.where(m, liota, -1.0)
    lmax = jnp.max(lsel, axis=-1, keepdims=True)
    maskf = (lmax >= 0.0).astype(f32)
    wsel = (m & (lsel == lmax)).astype(f32)
    mtbx = jnp.sum(wsel * adjx, axis=-1, keepdims=True)
    mtby = jnp.sum(wsel * adjy, axis=-1, keepdims=True)
    mtbw = jnp.sum(wsel * adjw, axis=-1, keepdims=True)
    mtbh = jnp.sum(wsel * adjh, axis=-1, keepdims=True)
    mtbc = jnp.sum(wsel * gc, axis=-1, keepdims=True)

    mx = jnp.max(tcls, axis=-1, keepdims=True)
    e = jnp.exp(tcls - mx)
    se = jnp.sum(e, axis=-1, keepdims=True)
    p = e / se
    pconf = jax.nn.sigmoid(tc)

    no_obj = (1.0 - obj_det) * (1.0 - maskf) * (pconf * pconf)
    obj = 5.0 * maskf * (1.0 - pconf) ** 2
    onehot = (lax.broadcasted_iota(jnp.int32, (1, _C), 1).astype(f32)
              == mtbc).astype(f32)
    cls_l = maskf * (onehot - p) ** 2
    coord = maskf * ((mtbx - sx) ** 2 + (mtby - sy) ** 2
                     + (mtbw - tw) ** 2 + (mtbh - th) ** 2)
    total = 0.5 * (jnp.sum(no_obj) + jnp.sum(obj)
                   + jnp.sum(cls_l) + jnp.sum(coord))
    loss_ref[:, :, :] = loss_ref[:, :, :] + total

    boxes = jnp.concatenate(
        [pminx * _IMG, pminy * _IMG, pmaxx * _IMG, pmaxy * _IMG], axis=-1)
    boxes_ref[0, 0] = boxes
    scores_ref[0, 0] = pconf * p


def _yolo_kernel(anchors_ref, p0_ref, p1_ref, p2_ref, gt_ref,
                 loss_ref, boxes_ref, scores_ref):
    n = pl.program_id(1)

    @pl.when(n == 0)
    def _init():
        loss_ref[:, :, :] = jnp.zeros_like(loss_ref)

    g = gt_ref[0]

    @pl.when(n < _SEGS[0])
    def _l0():
        _level_body(p0_ref[0, 0], g, anchors_ref, n, 60,
                    loss_ref, boxes_ref, scores_ref)

    @pl.when((n >= _SEGS[0]) & (n < _SEGS[0] + _SEGS[1]))
    def _l1():
        _level_body(p1_ref[0, 0], g, anchors_ref, n - _SEGS[0], 30,
                    loss_ref, boxes_ref, scores_ref)

    @pl.when(n == _SEGS[0] + _SEGS[1])
    def _l2():
        _level_body(p2_ref[0, 0], g, anchors_ref, n - _SEGS[0] - _SEGS[1], 15,
                    loss_ref, boxes_ref, scores_ref)


def kernel(preds0, preds1, preds2, gt_labels, anchors):
    f32 = jnp.float32
    p0 = preds0.reshape(_B, _SEGS[0], _R, 85)
    p1 = preds1.reshape(_B, _SEGS[1], _R, 85)
    p2 = preds2.reshape(_B, _SEGS[2], _R, 85)
    gt_t = jnp.transpose(gt_labels, (0, 2, 1))

    loss_p, boxes, scores = pl.pallas_call(
        _yolo_kernel,
        grid=(_B, _NSEG),
        in_specs=[
            pl.BlockSpec(memory_space=pltpu.SMEM),
            pl.BlockSpec((1, 1, _R, 85),
                         lambda b, n: (b, jnp.minimum(n, _SEGS[0] - 1), 0, 0)),
            pl.BlockSpec((1, 1, _R, 85),
                         lambda b, n: (b, jnp.clip(n - _SEGS[0], 0, _SEGS[1] - 1), 0, 0)),
            pl.BlockSpec((1, 1, _R, 85), lambda b, n: (b, 0, 0, 0)),
            pl.BlockSpec((1, 5, _L), lambda b, n: (b, 0, 0)),
        ],
        out_specs=[
            pl.BlockSpec((1, 1, 128), lambda b, n: (b, 0, 0)),
            pl.BlockSpec((1, 1, _R, 4), lambda b, n: (b, n, 0, 0)),
            pl.BlockSpec((1, 1, _R, _C), lambda b, n: (b, n, 0, 0)),
        ],
        out_shape=[
            jax.ShapeDtypeStruct((_B, 1, 128), f32),
            jax.ShapeDtypeStruct((_B, _NSEG, _R, 4), f32),
            jax.ShapeDtypeStruct((_B, _NSEG, _R, _C), f32),
        ],
        compiler_params=pltpu.CompilerParams(
            dimension_semantics=("parallel", "arbitrary")),
    )(anchors, p0, p1, p2, gt_t)

    loss = jnp.sum(loss_p[:, 0, 0])
    return loss, boxes.reshape(_B, _NSEG * _R, 4), scores.reshape(_B, _NSEG * _R, _C)


# speedup vs baseline: 4.4978x; 1.8170x over previous
import jax
import jax.numpy as jnp
from jax import lax
from jax.experimental import pallas as pl
from jax.experimental.pallas import tpu as pltpu

_B, _L, _C, _A = 32, 50, 80, 3
_IMG = 480.0
_CH = 5 + _C
_CHUNK = 1024
_LEVELS = (
    (60, 10800, 11),
    (30, 2700, 3),
    (15, 675, 1),
)


def _make_level_kernel(W, N, nch):
    Wf = float(W)
    f32 = jnp.float32

    def kern(anchors_ref, x_ref, gt_ref, loss_ref, boxes_ref, scores_ref):
        c = pl.program_id(1)

        @pl.when(c == 0)
        def _init():
            loss_ref[:, :, :] = jnp.zeros_like(loss_ref)

        ch = lambda k: x_ref[0, k, 0]

        rows = (c * _CHUNK
                + lax.broadcasted_iota(jnp.int32, (8, 128), 0) * 128
                + lax.broadcasted_iota(jnp.int32, (8, 128), 1)).astype(f32)
        cell = jnp.floor((rows + 0.5) * (1.0 / 3.0))
        a = rows - 3.0 * cell
        iF = jnp.floor((cell + 0.5) / Wf)
        jF = cell - Wf * iF
        validc = (rows < float(N)).astype(f32)

        gt = gt_ref[0]
        gx = gt[:, 0:1]
        gy = gt[:, 1:2]
        gw = gt[:, 2:3]
        gh = gt[:, 3:4]
        gc = gt[:, 4:5]
        bw = gw * Wf
        bh = gh * Wf
        validg = bw > 0.0
        jg = jnp.clip(jnp.floor(gx * Wf), 0.0, Wf - 1.0)
        ig = jnp.clip(jnp.floor(gy * Wf), 0.0, Wf - 1.0)
        aw = [anchors_ref[k, 0] * Wf for k in range(_A)]
        ah = [anchors_ref[k, 1] * Wf for k in range(_A)]

        def anc_iou(k):
            inter = jnp.minimum(bw, aw[k]) * jnp.minimum(bh, ah[k])
            return inter / (bw * bh + aw[k] * ah[k] - inter + 1e-9)

        kb = jnp.zeros_like(gx)
        bestk = anc_iou(0)
        for k in (1, 2):
            iouk = anc_iou(k)
            upd = iouk > bestk
            kb = jnp.where(upd, float(k), kb)
            bestk = jnp.where(upd, iouk, bestk)
        anc_w = jnp.where(kb == 0.0, aw[0], jnp.where(kb == 1.0, aw[1], aw[2]))
        anc_h = jnp.where(kb == 0.0, ah[0], jnp.where(kb == 1.0, ah[1], ah[2]))
        bw_s = jnp.where(validg, bw, 1.0)
        bh_s = jnp.where(validg, bh, 1.0)

        bc = lambda v: jnp.broadcast_to(v, (_L, 128))
        jg_m = bc(jnp.where(validg, jg, -1.0))
        ig_b = bc(ig)
        kb_b = bc(kb)
        adjx = bc(gx * Wf - jg)
        adjy = bc(gy * Wf - ig)
        adjw = bc(jnp.log(bw_s / anc_w))
        adjh = bc(jnp.log(bh_s / anc_h))
        gc_b = bc(gc)
        tminx = bc(gx - gw * 0.5)
        tmaxx = bc(gx + gw * 0.5)
        tminy = bc(gy - gh * 0.5)
        tmaxy = bc(gy + gh * 0.5)
        tarea = bc(gw * gh)

        tw = ch(2)
        th = ch(3)
        sx = jax.nn.sigmoid(ch(0))
        sy = jax.nn.sigmoid(ch(1))
        pconf = jax.nn.sigmoid(ch(4))
        aw_c = jnp.where(a == 0.0, aw[0], jnp.where(a == 1.0, aw[1], aw[2]))
        ah_c = jnp.where(a == 0.0, ah[0], jnp.where(a == 1.0, ah[1], ah[2]))
        px = (sx + jF) / Wf
        py = (sy + iF) / Wf
        pw = jnp.exp(tw) * aw_c / Wf
        ph = jnp.exp(th) * ah_c / Wf
        pminx = px - pw * 0.5
        pmaxx = px + pw * 0.5
        pminy = py - ph * 0.5
        pmaxy = py + ph * 0.5
        parea = pw * ph

        best = jnp.zeros((8, 128), f32)
        maskf = jnp.zeros((8, 128), f32)
        mtbx = jnp.zeros((8, 128), f32)
        mtby = jnp.zeros((8, 128), f32)
        mtbw = jnp.zeros((8, 128), f32)
        mtbh = jnp.zeros((8, 128), f32)
        mtbc = jnp.zeros((8, 128), f32)
        for l in range(_L):
            r = lambda q: q[l:l + 1, :]
            iw = jnp.clip(jnp.minimum(pmaxx, r(tmaxx))
                          - jnp.maximum(pminx, r(tminx)), 0.0)
            ih = jnp.clip(jnp.minimum(pmaxy, r(tmaxy))
                          - jnp.maximum(pminy, r(tminy)), 0.0)
            inter = iw * ih
            iou = inter / (parea + r(tarea) - inter + 1e-9)
            best = jnp.maximum(best, iou)
            m = (jF == r(jg_m)) & (iF == r(ig_b)) & (a == r(kb_b))
            maskf = jnp.where(m, 1.0, maskf)
            mtbx = jnp.where(m, r(adjx), mtbx)
            mtby = jnp.where(m, r(adjy), mtby)
            mtbw = jnp.where(m, r(adjw), mtbw)
            mtbh = jnp.where(m, r(adjh), mtbh)
            mtbc = jnp.where(m, r(gc_b), mtbc)
        obj_det = (best > 0.6).astype(f32)

        mx = ch(5)
        for k in range(6, _CH):
            mx = jnp.maximum(mx, ch(k))
        se = jnp.zeros((8, 128), f32)
        for k in range(_C):
            se = se + jnp.exp(ch(5 + k) - mx)
        rse = 1.0 / se
        cls_acc = jnp.zeros((8, 128), f32)
        for k in range(_C):
            p = jnp.exp(ch(5 + k) - mx) * rse
            scores_ref[0, k, 0] = pconf * p
            oh = (mtbc == float(k)).astype(f32)
            d = oh - p
            cls_acc = cls_acc + d * d
        cls_acc = cls_acc * maskf

        no_obj = (1.0 - obj_det) * (1.0 - maskf) * (pconf * pconf) * validc
        obj = 5.0 * maskf * (1.0 - pconf) ** 2
        coord = maskf * ((mtbx - sx) ** 2 + (mtby - sy) ** 2
                         + (mtbw - tw) ** 2 + (mtbh - th) ** 2)
        total = 0.5 * jnp.sum(no_obj + obj + coord + cls_acc)
        loss_ref[:, :, :] = loss_ref[:, :, :] + total

        boxes_ref[0, 0, 0] = pminx * _IMG
        boxes_ref[0, 1, 0] = pminy * _IMG
        boxes_ref[0, 2, 0] = pmaxx * _IMG
        boxes_ref[0, 3, 0] = pmaxy * _IMG

    return kern


def _run_level(preds, gt_labels, anchors, W, N, nch):
    f32 = jnp.float32
    npad = nch * _CHUNK
    pt = preds.reshape(_B, N, _CH).transpose(0, 2, 1)
    pt = jnp.pad(pt, ((0, 0), (0, 0), (0, npad - N)))
    pt = pt.reshape(_B, _CH, nch, 8, 128)

    loss_p, boxes_t, scores_t = pl.pallas_call(
        _make_level_kernel(W, N, nch),
        grid=(_B, nch),
        in_specs=[
            pl.BlockSpec(memory_space=pltpu.SMEM),
            pl.BlockSpec((1, _CH, 1, 8, 128), lambda b, c: (b, 0, c, 0, 0)),
            pl.BlockSpec((1, _L, 5), lambda b, c: (b, 0, 0)),
        ],
        out_specs=[
            pl.BlockSpec((1, 1, 128), lambda b, c: (b, 0, 0)),
            pl.BlockSpec((1, 4, 1, 8, 128), lambda b, c: (b, 0, c, 0, 0)),
            pl.BlockSpec((1, _C, 1, 8, 128), lambda b, c: (b, 0, c, 0, 0)),
        ],
        out_shape=[
            jax.ShapeDtypeStruct((_B, 1, 128), f32),
            jax.ShapeDtypeStruct((_B, 4, nch, 8, 128), f32),
            jax.ShapeDtypeStruct((_B, _C, nch, 8, 128), f32),
        ],
        compiler_params=pltpu.CompilerParams(
            dimension_semantics=("parallel", "arbitrary")),
    )(anchors, pt, gt_labels)

    return (loss_p[:, 0, 0],
            boxes_t.reshape(_B, 4, npad)[:, :, :N],
            scores_t.reshape(_B, _C, npad)[:, :, :N])


def kernel(preds0, preds1, preds2, gt_labels, anchors):
    losses, boxes_l, scores_l = [], [], []
    for preds, (W, N, nch) in zip((preds0, preds1, preds2), _LEVELS):
        lp, bx, sc = _run_level(preds, gt_labels, anchors, W, N, nch)
        losses.append(lp)
        boxes_l.append(bx)
        scores_l.append(sc)
    loss = sum(jnp.sum(lp) for lp in losses)
    boxes = jnp.concatenate(boxes_l, axis=2).transpose(0, 2, 1)
    scores = jnp.concatenate(scores_l, axis=2).transpose(0, 2, 1)
    return loss, boxes, scores
